# baseline (speedup 1.0000x reference)
import functools

import jax
import jax.numpy as jnp
from jax import lax
from jax.experimental import pallas as pl
from jax.experimental.pallas import tpu as pltpu
from jax.experimental.pallas import tpu_sc as plsc

_N, _E, _D, _H, _HID = 10000, 320000, 128, 4, 128
_N2 = 10240
_NC, _NS, _L = 2, 16, 16
_NW = _NC * _NS
_B = 80
_EPW = _E // _NW
_NCH = _EPW // _B
_RPT = _N2 // _NS
_CR = _N2 // _HID
_RB = 512
_XWW = _H * _HID

_mesh = plsc.VectorSubcoreMesh(core_axis_name="c", subcore_axis_name="s")
_sc_params = pltpu.CompilerParams(needs_layout_passes=False)


def _make_q_kernel(with_cnt):
    out_type = [jax.ShapeDtypeStruct((_E * _H,), jnp.float32)]
    bq = _B if with_cnt else 400
    nch = _EPW // bq
    scratch = [
        pltpu.VMEM((bq,), jnp.int32),
        pltpu.VMEM((bq,), jnp.int32),
        pltpu.VMEM((bq,), jnp.int32),
        pltpu.VMEM((bq,), jnp.int32),
        pltpu.VMEM((_N2 * _H,), jnp.float32),
        pltpu.VMEM((bq * _H,), jnp.float32),
        pltpu.VMEM((bq * _H,), jnp.float32),
        pltpu.VMEM((16,), jnp.float32),
        pltpu.SemaphoreType.DMA,
        pltpu.SemaphoreType.DMA,
        pltpu.SemaphoreType.DMA,
        pltpu.SemaphoreType.DMA,
    ]
    if with_cnt:
        out_type.append(jax.ShapeDtypeStruct((_NC * _CR, _HID), jnp.float32))
        scratch += [
            pltpu.VMEM_SHARED((_CR, _HID), jnp.float32),
            pltpu.VMEM((bq, _HID), jnp.float32),
            pltpu.VMEM((bq,), jnp.int32),
            pltpu.VMEM((bq,), jnp.int32),
        ]

    @functools.partial(
        pl.kernel,
        out_type=out_type,
        mesh=_mesh,
        compiler_params=_sc_params,
        scratch_types=scratch,
    )
    def q_kernel(src_hbm, dst_hbm, xu_hbm, c_hbm, *rest):
        if with_cnt:
            (q_hbm, cnt_hbm, src0_v, src1_v, dst0_v, dst1_v, xut_v,
             qo0_v, qo1_v, c_v, sem_i0, sem_i1, sem_q0, sem_q1,
             cnt_sh, oh_v, ddiv_v, pmod_v) = rest
        else:
            (q_hbm, src0_v, src1_v, dst0_v, dst1_v, xut_v,
             qo0_v, qo1_v, c_v, sem_i0, sem_i1, sem_q0, sem_q1) = rest
        cid = lax.axis_index("c")
        sid = lax.axis_index("s")
        wid = sid * _NC + cid

        zero16 = jnp.zeros((16,), jnp.float32)
        izero16 = jnp.zeros((16,), jnp.int32)
        one16 = jnp.ones((16,), jnp.float32)
        iota16 = lax.iota(jnp.int32, 16)

        pltpu.sync_copy(c_hbm, c_v)
        pltpu.sync_copy(xu_hbm, xut_v)
        cvec = c_v[...]
        if with_cnt:
            def _zoh(r, carry):
                for j in range(_HID // 16):
                    oh_v[r, pl.ds(j * 16, 16)] = zero16
                return carry

            lax.fori_loop(0, bq, _zoh, None)
            for t in range(bq // 16):
                pmod_v[pl.ds(t * 16, 16)] = izero16

            @pl.when(sid == 0)
            def _():
                pltpu.sync_copy(oh_v.at[pl.ds(0, _CR)], cnt_sh)

            plsc.subcore_barrier()

        ebase = wid * _EPW
        srcs = [src0_v, src1_v]
        dsts = [dst0_v, dst1_v]
        qos = [qo0_v, qo1_v]
        sem_i = [sem_i0, sem_i1]
        sem_q = [sem_q0, sem_q1]

        def _fetch(g, p):
            b = ebase + g * bq
            pltpu.async_copy(src_hbm.at[pl.ds(b, bq)], srcs[p], sem_i[p])
            pltpu.async_copy(dst_hbm.at[pl.ds(b, bq)], dsts[p], sem_i[p])

        def _fetch_wait(p):
            pltpu.make_async_copy(src_hbm.at[pl.ds(0, bq)], srcs[p],
                                  sem_i[p]).wait()
            pltpu.make_async_copy(dst_hbm.at[pl.ds(0, bq)], dsts[p],
                                  sem_i[p]).wait()

        def _qwrite_wait(p):
            pltpu.make_async_copy(qos[p], q_hbm.at[pl.ds(0, bq * _H)],
                                  sem_q[p]).wait()

        def _chunk(g, p, prefetch, first):
            if prefetch:
                _fetch(g + 1, 1 - p)
            _fetch_wait(p)
            if not first:
                _qwrite_wait(p)
            src_v = srcs[p]
            dst_v = dsts[p]
            qo_v = qos[p]
            for t in range(bq // 16):
                e16 = jnp.full((16,), t * 16, jnp.int32) + iota16
                srcv = src_v[pl.ds(t * 16, 16)]
                dstv = dst_v[pl.ds(t * 16, 16)]
                if with_cnt:
                    dmod = jnp.bitwise_and(dstv, _HID - 1)
                    ddiv_v[pl.ds(t * 16, 16)] = jnp.right_shift(dstv, 7)
                    pmod = pmod_v[pl.ds(t * 16, 16)]
                    plsc.store_scatter(oh_v, [e16, pmod], zero16)
                    plsc.store_scatter(oh_v, [e16, dmod], one16)
                    pmod_v[pl.ds(t * 16, 16)] = dmod
                srcu = srcv * _H
                dstu = dstv * _H
                zs = []
                for h in range(_H):
                    a = plsc.load_gather(xut_v, [srcu + h]) + cvec[h]
                    b = plsc.load_gather(xut_v, [dstu + h])
                    zs.append(a - b)
                mx = jnp.maximum(jnp.maximum(zs[0], zs[1]),
                                 jnp.maximum(zs[2], zs[3]))
                es = [jnp.exp(z - mx) for z in zs]
                tot = es[0] + es[1] + es[2] + es[3]
                e16h = e16 * _H
                for h in range(_H):
                    plsc.store_scatter(qo_v, [e16h + h], es[h] / tot)
            pltpu.async_copy(
                qo_v, q_hbm.at[pl.ds((ebase + g * bq) * _H, bq * _H)],
                sem_q[p])
            if with_cnt:
                pltpu.sync_copy(oh_v, cnt_sh.at[ddiv_v], add=True)

        _fetch(0, 0)
        _chunk(0, 0, True, True)
        _chunk(1, 1, True, True)

        def pair_body(i, carry):
            g = 2 + 2 * i
            _chunk(g, 0, True, False)
            _chunk(g + 1, 1, True, False)
            return carry

        lax.fori_loop(0, (nch - 3) // 2, pair_body, None)
        _chunk(nch - 1, (nch - 1) % 2, False, False)
        _qwrite_wait(0)
        _qwrite_wait(1)
        if with_cnt:
            plsc.subcore_barrier()

            @pl.when(sid == 0)
            def _():
                pltpu.sync_copy(cnt_sh, cnt_hbm.at[pl.ds(cid * _CR, _CR)])

    return q_kernel


_q_kernel_cnt = _make_q_kernel(True)
_q_kernel = _make_q_kernel(False)


@functools.partial(
    pl.kernel,
    out_type=[jax.ShapeDtypeStruct((_NC * _N2, _HID), jnp.float32)],
    mesh=_mesh,
    compiler_params=_sc_params,
    scratch_types=[
        pltpu.VMEM_SHARED((_N2, _HID), jnp.float32),
        pltpu.VMEM((_B,), jnp.int32),
        pltpu.VMEM((_B,), jnp.int32),
        pltpu.VMEM((_B,), jnp.int32),
        pltpu.VMEM((_B,), jnp.int32),
        pltpu.VMEM((_B * _H,), jnp.float32),
        pltpu.VMEM((_B * _H,), jnp.float32),
        pltpu.VMEM((_B, _HID), jnp.float32),
        pltpu.VMEM((_B, _HID), jnp.float32),
        pltpu.VMEM((_B, _HID), jnp.float32),
        pltpu.VMEM((_B, _HID), jnp.float32),
        pltpu.VMEM((_B,), jnp.int32),
        pltpu.VMEM((_B,), jnp.int32),
        pltpu.SemaphoreType.DMA,
        pltpu.SemaphoreType.DMA,
        pltpu.SemaphoreType.DMA,
        pltpu.SemaphoreType.DMA,
        pltpu.SemaphoreType.DMA,
        pltpu.SemaphoreType.DMA,
    ],
)
def _agg_kernel(src_hbm, dst_hbm, q_hbm, t01_hbm, t23_hbm,
                out_hbm, acc_sh, src0_v, src1_v, dst0_v, dst1_v, q0_v, q1_v,
                rows_0, rows_1, m0_v, m1_v,
                sd0_v, sd1_v, sem_0, sem_1, sem_i0, sem_i1, sem_s0, sem_s1):
    cid = lax.axis_index("c")
    sid = lax.axis_index("s")
    wid = sid * _NC + cid

    zero16 = jnp.zeros((16,), jnp.float32)
    ones16 = jnp.full((16,), 1, jnp.int32)

    def _zrow(r, carry):
        for j in range(_HID // 16):
            m0_v[r, pl.ds(j * 16, 16)] = zero16
        return carry

    lax.fori_loop(0, _B, _zrow, None)
    row0 = sid * _RPT
    for k in range(_RPT // _B):
        pltpu.sync_copy(m0_v, acc_sh.at[pl.ds(row0 + k * _B, _B)])
    plsc.subcore_barrier()

    ebase = wid * _EPW
    tabs = [t01_hbm, t23_hbm]
    bufs = [rows_0, rows_1]
    sems = [sem_0, sem_1]
    sem_i = [sem_i0, sem_i1]
    sem_s = [sem_s0, sem_s1]
    srcs = [src0_v, src1_v]
    dsts = [dst0_v, dst1_v]
    qs = [q0_v, q1_v]
    ms = [m0_v, m1_v]
    sd = [sd0_v, sd1_v]

    def _idx_fetch(g, p):
        b = ebase + g * _B
        pltpu.async_copy(src_hbm.at[pl.ds(b, _B)], srcs[p], sem_i[p])
        pltpu.async_copy(dst_hbm.at[pl.ds(b, _B)], dsts[p], sem_i[p])
        pltpu.async_copy(q_hbm.at[pl.ds(b * _H, _B * _H)], qs[p], sem_i[p])

    def _idx_wait(p):
        pltpu.make_async_copy(src_hbm.at[pl.ds(0, _B)], srcs[p],
                              sem_i[p]).wait()
        pltpu.make_async_copy(dst_hbm.at[pl.ds(0, _B)], dsts[p],
                              sem_i[p]).wait()
        pltpu.make_async_copy(q_hbm.at[pl.ds(0, _B * _H)], qs[p],
                              sem_i[p]).wait()

    def _scatter_wait(p):
        pltpu.make_async_copy(ms[p], acc_sh.at[sd[p]], sem_s[p]).wait()

    def _chunk(g, p, prefetch, first):
        q_v = qs[p]
        m_v = ms[p]
        if prefetch:
            _idx_fetch(g + 1, 1 - p)
        for ph in range(2):
            pltpu.make_async_copy(tabs[ph].at[srcs[p]], bufs[ph],
                                  sems[ph]).wait()
            if ph == 0 and not first:
                _scatter_wait(p)
            buf = bufs[ph]

            def edge(e, ph=ph, buf=buf, q_v=q_v, m_v=m_v):
                qa = plsc.load_gather(q_v, [ones16 * (e * _H + 2 * ph)])
                qb = plsc.load_gather(q_v, [ones16 * (e * _H + 2 * ph + 1)])
                for j in range(_HID // 16):
                    a, b = plsc.unpack(
                        plsc.bitcast(buf[e, pl.ds(j * 16, 16)],
                                     jnp.bfloat16),
                        format=plsc.PackFormat.INTERLEAVED)
                    v = qa * a + qb * b
                    if ph > 0:
                        v = v + m_v[e, pl.ds(j * 16, 16)]
                    m_v[e, pl.ds(j * 16, 16)] = v

            plsc.parallel_loop(0, _B, unroll=4)(edge)
            if prefetch:
                if ph == 0:
                    _idx_wait(1 - p)
                pltpu.async_copy(tabs[ph].at[srcs[1 - p]], bufs[ph],
                                 sems[ph])
        for t in range(_B // 16):
            sd[p][pl.ds(t * 16, 16)] = dsts[p][pl.ds(t * 16, 16)]
        pltpu.async_copy(m_v, acc_sh.at[sd[p]], sem_s[p], add=True)

    _idx_fetch(0, 0)
    _idx_wait(0)
    pltpu.async_copy(tabs[0].at[srcs[0]], bufs[0], sems[0])
    pltpu.async_copy(tabs[1].at[srcs[0]], bufs[1], sems[1])

    _chunk(0, 0, True, True)
    _chunk(1, 1, True, True)

    def pair_body(i, carry):
        g = 2 + 2 * i
        _chunk(g, 0, True, False)
        _chunk(g + 1, 1, True, False)
        return carry

    lax.fori_loop(0, (_NCH - 3) // 2, pair_body, None)
    _chunk(_NCH - 1, (_NCH - 1) % 2, False, False)
    _scatter_wait(0)
    _scatter_wait(1)
    plsc.subcore_barrier()

    for k in range(_RPT // _B):
        r = row0 + k * _B
        pltpu.sync_copy(acc_sh.at[pl.ds(r, _B)],
                        out_hbm.at[pl.ds(cid * _N2 + r, _B)])


def _pack2(lo, hi):
    l16 = jax.lax.bitcast_convert_type(lo.astype(jnp.bfloat16),
                                       jnp.uint16).astype(jnp.uint32)
    h16 = jax.lax.bitcast_convert_type(hi.astype(jnp.bfloat16),
                                       jnp.uint16).astype(jnp.uint32)
    return jax.lax.bitcast_convert_type(l16 | (h16 << 16), jnp.float32)


def _head_tables(v, wc_ref, t01_ref, t23_ref):
    hd = [jnp.dot(v, wc_ref[:, i * _HID:(i + 1) * _HID],
                  preferred_element_type=jnp.float32) for i in range(_H)]
    t01_ref[...] = _pack2(hd[0], hd[1])
    t23_ref[...] = _pack2(hd[2], hd[3])


def _tc_pre_body(x_ref, w1_ref, b1_ref, w2_ref, b2_ref, u1_ref, wc_ref,
                 t01_ref, t23_ref, xu_ref, sk_ref):
    x = x_ref[...]
    xp = jnp.dot(x, w1_ref[...], preferred_element_type=jnp.float32) + b1_ref[...]
    _head_tables(xp, wc_ref, t01_ref, t23_ref)
    xu_ref[...] = jnp.dot(xp, u1_ref[...], preferred_element_type=jnp.float32)
    sk_ref[...] = jnp.dot(x, w2_ref[...], preferred_element_type=jnp.float32) + b2_ref[...]


def _tc_mid_body(a0_ref, a1_ref, cnt_ref, sk1_ref, bias1_ref, u2_ref,
                 wc_ref, w3_ref, b3_ref,
                 t01_ref, t23_ref, xu_ref, sk2_ref, inv_ref):
    s = a0_ref[...] + a1_ref[...]
    inv = 1.0 / jnp.maximum(cnt_ref[...][:, 0:1], 1.0)
    h = jnp.maximum(s * inv + bias1_ref[...] + sk1_ref[...], 0.0)
    _head_tables(h, wc_ref, t01_ref, t23_ref)
    xu_ref[...] = jnp.dot(h, u2_ref[...], preferred_element_type=jnp.float32)
    sk2_ref[...] = jnp.dot(h, w3_ref[...], preferred_element_type=jnp.float32) + b3_ref[...]
    inv_ref[...] = jnp.broadcast_to(inv, inv_ref.shape)


def _tc_post_body(a0_ref, a1_ref, sk2_ref, inv_ref, bias2_ref, out_ref):
    s = (a0_ref[...] + a1_ref[...]) * inv_ref[...][:, 0:1]
    out_ref[...] = jnp.maximum(s + bias2_ref[...] + sk2_ref[...], 0.0)


def _row_spec(w):
    return pl.BlockSpec((_RB, w), lambda i: (i, 0))


def _full_spec(r, c):
    return pl.BlockSpec((r, c), lambda i: (0, 0))


_GRID = (_N2 // _RB,)
_TABF = jax.ShapeDtypeStruct((_N2, _HID), jnp.float32)

_tc_pre = pl.pallas_call(
    _tc_pre_body,
    grid=_GRID,
    in_specs=[_row_spec(_D), _full_spec(_D, _D), _full_spec(1, _D),
              _full_spec(_D, _D), _full_spec(1, _D), _full_spec(_D, 16),
              _full_spec(_D, _XWW)],
    out_specs=[_row_spec(_HID)] * 2 + [_row_spec(16), _row_spec(_HID)],
    out_shape=[_TABF] * 2 + [jax.ShapeDtypeStruct((_N2, 16), jnp.float32),
                             _TABF],
)

_tc_mid = pl.pallas_call(
    _tc_mid_body,
    grid=_GRID,
    in_specs=[_row_spec(_HID), _row_spec(_HID), _row_spec(8), _row_spec(_HID),
              _full_spec(1, _HID), _full_spec(_HID, 16),
              _full_spec(_HID, _XWW),
              _full_spec(_HID, _HID), _full_spec(1, _HID)],
    out_specs=[_row_spec(_HID)] * 2 + [_row_spec(16), _row_spec(_HID),
                                       _row_spec(8)],
    out_shape=[_TABF] * 2 + [jax.ShapeDtypeStruct((_N2, 16), jnp.float32),
                             _TABF,
                             jax.ShapeDtypeStruct((_N2, 8), jnp.float32)],
)

_tc_post = pl.pallas_call(
    _tc_post_body,
    grid=_GRID,
    in_specs=[_row_spec(_HID), _row_spec(_HID), _row_spec(_HID),
              _row_spec(8), _full_spec(1, _HID)],
    out_specs=_row_spec(_HID),
    out_shape=jax.ShapeDtypeStruct((_N2, _HID), jnp.float32),
)


def kernel(x, edge_index, W1, b1, W2, b2, W3, b3, U1, c1, Wc1, bias1,
           U2, c2, Wc2, bias2):
    xpad = jnp.pad(x, ((0, _N2 - _N), (0, 0)))
    src = edge_index[0]
    dst = edge_index[1]
    u1p = jnp.pad(U1, ((0, 0), (0, 16 - _H)))
    u2p = jnp.pad(U2, ((0, 0), (0, 16 - _H)))
    c1p = jnp.pad(c1, (0, 16 - _H))
    c2p = jnp.pad(c2, (0, 16 - _H))

    t01a, t23a, xu1, sk1 = _tc_pre(
        xpad, W1, b1.reshape(1, -1), W2, b2.reshape(1, -1), u1p, Wc1)
    q1, cnt1 = _q_kernel_cnt(src, dst, xu1[:, :_H].reshape(-1), c1p)
    acc1, = _agg_kernel(src, dst, q1, t01a, t23a)
    cnt8 = jnp.broadcast_to(
        (cnt1[:_CR] + cnt1[_CR:]).reshape(_N2, 1), (_N2, 8))
    t01b, t23b, xu2, sk2, inv = _tc_mid(
        acc1[:_N2], acc1[_N2:], cnt8, sk1, bias1.reshape(1, -1), u2p,
        Wc2, W3, b3.reshape(1, -1))
    q2, = _q_kernel(src, dst, xu2[:, :_H].reshape(-1), c2p)
    acc2, = _agg_kernel(src, dst, q2, t01b, t23b)
    out = _tc_post(acc2[:_N2], acc2[_N2:], sk2, inv, bias2.reshape(1, -1))
    return out[:_N]


# speedup vs baseline: 1.1749x; 1.1749x over previous
import functools

import jax
import jax.numpy as jnp
from jax import lax
from jax.experimental import pallas as pl
from jax.experimental.pallas import tpu as pltpu
from jax.experimental.pallas import tpu_sc as plsc

_N, _E, _D, _H, _HID = 10000, 320000, 128, 4, 128
_N2 = 10240
_NC, _NS, _L = 2, 16, 16
_NW = _NC * _NS
_B = 80
_EPW = _E // _NW
_NCH = _EPW // _B
_RPT = _N2 // _NS
_CR = _N2 // _HID
_RB = 512
_XWW = _H * _HID

_mesh = plsc.VectorSubcoreMesh(core_axis_name="c", subcore_axis_name="s")
_sc_params = pltpu.CompilerParams(needs_layout_passes=False)


def _make_q_kernel(with_cnt):
    out_type = [jax.ShapeDtypeStruct((_E * _H,), jnp.float32)]
    bq = _B if with_cnt else 400
    nch = _EPW // bq
    scratch = [
        pltpu.VMEM((bq,), jnp.int32),
        pltpu.VMEM((bq,), jnp.int32),
        pltpu.VMEM((bq,), jnp.int32),
        pltpu.VMEM((bq,), jnp.int32),
        pltpu.VMEM((_N2 * _H,), jnp.float32),
        pltpu.VMEM((bq * _H,), jnp.float32),
        pltpu.VMEM((bq * _H,), jnp.float32),
        pltpu.VMEM((16,), jnp.float32),
        pltpu.SemaphoreType.DMA,
        pltpu.SemaphoreType.DMA,
        pltpu.SemaphoreType.DMA,
        pltpu.SemaphoreType.DMA,
    ]
    if with_cnt:
        out_type.append(jax.ShapeDtypeStruct((_NC * _CR, _HID), jnp.float32))
        scratch += [
            pltpu.VMEM_SHARED((_CR, _HID), jnp.float32),
            pltpu.VMEM((bq, _HID), jnp.float32),
            pltpu.VMEM((bq,), jnp.int32),
            pltpu.VMEM((bq,), jnp.int32),
        ]

    @functools.partial(
        pl.kernel,
        out_type=out_type,
        mesh=_mesh,
        compiler_params=_sc_params,
        scratch_types=scratch,
    )
    def q_kernel(src_hbm, dst_hbm, xu_hbm, c_hbm, *rest):
        if with_cnt:
            (q_hbm, cnt_hbm, src0_v, src1_v, dst0_v, dst1_v, xut_v,
             qo0_v, qo1_v, c_v, sem_i0, sem_i1, sem_q0, sem_q1,
             cnt_sh, oh_v, ddiv_v, pmod_v) = rest
        else:
            (q_hbm, src0_v, src1_v, dst0_v, dst1_v, xut_v,
             qo0_v, qo1_v, c_v, sem_i0, sem_i1, sem_q0, sem_q1) = rest
        cid = lax.axis_index("c")
        sid = lax.axis_index("s")
        wid = sid * _NC + cid

        zero16 = jnp.zeros((16,), jnp.float32)
        izero16 = jnp.zeros((16,), jnp.int32)
        one16 = jnp.ones((16,), jnp.float32)
        iota16 = lax.iota(jnp.int32, 16)

        pltpu.sync_copy(c_hbm, c_v)
        pltpu.sync_copy(xu_hbm, xut_v)
        cvec = c_v[...]
        if with_cnt:
            def _zoh(r, carry):
                for j in range(_HID // 16):
                    oh_v[r, pl.ds(j * 16, 16)] = zero16
                return carry

            lax.fori_loop(0, bq, _zoh, None)
            for t in range(bq // 16):
                pmod_v[pl.ds(t * 16, 16)] = izero16

            @pl.when(sid == 0)
            def _():
                pltpu.sync_copy(oh_v.at[pl.ds(0, _CR)], cnt_sh)

            plsc.subcore_barrier()

        ebase = wid * _EPW
        srcs = [src0_v, src1_v]
        dsts = [dst0_v, dst1_v]
        qos = [qo0_v, qo1_v]
        sem_i = [sem_i0, sem_i1]
        sem_q = [sem_q0, sem_q1]

        def _fetch(g, p):
            b = ebase + g * bq
            pltpu.async_copy(src_hbm.at[pl.ds(b, bq)], srcs[p], sem_i[p])
            pltpu.async_copy(dst_hbm.at[pl.ds(b, bq)], dsts[p], sem_i[p])

        def _fetch_wait(p):
            pltpu.make_async_copy(src_hbm.at[pl.ds(0, bq)], srcs[p],
                                  sem_i[p]).wait()
            pltpu.make_async_copy(dst_hbm.at[pl.ds(0, bq)], dsts[p],
                                  sem_i[p]).wait()

        def _qwrite_wait(p):
            pltpu.make_async_copy(qos[p], q_hbm.at[pl.ds(0, bq * _H)],
                                  sem_q[p]).wait()

        def _chunk(g, p, prefetch, first):
            if prefetch:
                _fetch(g + 1, 1 - p)
            _fetch_wait(p)
            if not first:
                _qwrite_wait(p)
            src_v = srcs[p]
            dst_v = dsts[p]
            qo_v = qos[p]
            for t in range(bq // 16):
                e16 = jnp.full((16,), t * 16, jnp.int32) + iota16
                srcv = src_v[pl.ds(t * 16, 16)]
                dstv = dst_v[pl.ds(t * 16, 16)]
                if with_cnt:
                    dmod = jnp.bitwise_and(dstv, _HID - 1)
                    ddiv_v[pl.ds(t * 16, 16)] = jnp.right_shift(dstv, 7)
                    pmod = pmod_v[pl.ds(t * 16, 16)]
                    plsc.store_scatter(oh_v, [e16, pmod], zero16)
                    plsc.store_scatter(oh_v, [e16, dmod], one16)
                    pmod_v[pl.ds(t * 16, 16)] = dmod
                srcu = srcv * _H
                dstu = dstv * _H
                zs = []
                for h in range(_H):
                    a = plsc.load_gather(xut_v, [srcu + h]) + cvec[h]
                    b = plsc.load_gather(xut_v, [dstu + h])
                    zs.append(a - b)
                mx = jnp.maximum(jnp.maximum(zs[0], zs[1]),
                                 jnp.maximum(zs[2], zs[3]))
                es = [jnp.exp(z - mx) for z in zs]
                tot = es[0] + es[1] + es[2] + es[3]
                e16h = e16 * _H
                for h in range(_H):
                    plsc.store_scatter(qo_v, [e16h + h], es[h] / tot)
            pltpu.async_copy(
                qo_v, q_hbm.at[pl.ds((ebase + g * bq) * _H, bq * _H)],
                sem_q[p])
            if with_cnt:
                pltpu.sync_copy(oh_v, cnt_sh.at[ddiv_v], add=True)

        _fetch(0, 0)
        _chunk(0, 0, True, True)
        _chunk(1, 1, True, True)

        def pair_body(i, carry):
            g = 2 + 2 * i
            _chunk(g, 0, True, False)
            _chunk(g + 1, 1, True, False)
            return carry

        lax.fori_loop(0, (nch - 3) // 2, pair_body, None)
        _chunk(nch - 1, (nch - 1) % 2, False, False)
        _qwrite_wait(0)
        _qwrite_wait(1)
        if with_cnt:
            plsc.subcore_barrier()

            @pl.when(sid == 0)
            def _():
                pltpu.sync_copy(cnt_sh, cnt_hbm.at[pl.ds(cid * _CR, _CR)])

    return q_kernel


_q_kernel_cnt = _make_q_kernel(True)
_q_kernel = _make_q_kernel(False)


@functools.partial(
    pl.kernel,
    out_type=[jax.ShapeDtypeStruct((_NC * _N2, _HID), jnp.float32)],
    mesh=_mesh,
    compiler_params=_sc_params,
    scratch_types=[
        pltpu.VMEM_SHARED((_N2, _HID), jnp.float32),
        pltpu.VMEM((_B,), jnp.int32),
        pltpu.VMEM((_B,), jnp.int32),
        pltpu.VMEM((_B,), jnp.int32),
        pltpu.VMEM((_B,), jnp.int32),
        pltpu.VMEM((_B * _H,), jnp.float32),
        pltpu.VMEM((_B * _H,), jnp.float32),
        pltpu.VMEM((_B, _HID), jnp.float32),
        pltpu.VMEM((_B, _HID), jnp.float32),
        pltpu.VMEM((_B, _HID), jnp.float32),
        pltpu.VMEM((_B, _HID), jnp.float32),
        pltpu.VMEM((_B,), jnp.int32),
        pltpu.VMEM((_B,), jnp.int32),
        pltpu.SemaphoreType.DMA,
        pltpu.SemaphoreType.DMA,
        pltpu.SemaphoreType.DMA,
        pltpu.SemaphoreType.DMA,
        pltpu.SemaphoreType.DMA,
        pltpu.SemaphoreType.DMA,
    ],
)
def _agg_kernel(src_hbm, dst_hbm, q_hbm, t01_hbm, t23_hbm,
                out_hbm, acc_sh, src0_v, src1_v, dst0_v, dst1_v, q0_v, q1_v,
                rows_0, rows_1, m0_v, m1_v,
                sd0_v, sd1_v, sem_0, sem_1, sem_i0, sem_i1, sem_s0, sem_s1):
    cid = lax.axis_index("c")
    sid = lax.axis_index("s")
    wid = sid * _NC + cid

    zero16 = jnp.zeros((16,), jnp.float32)
    ones16 = jnp.full((16,), 1, jnp.int32)

    def _zrow(r, carry):
        for j in range(_HID // 16):
            m0_v[r, pl.ds(j * 16, 16)] = zero16
        return carry

    lax.fori_loop(0, _B, _zrow, None)
    row0 = sid * _RPT
    for k in range(_RPT // _B):
        pltpu.sync_copy(m0_v, acc_sh.at[pl.ds(row0 + k * _B, _B)])
    plsc.subcore_barrier()

    ebase = wid * _EPW
    tabs = [t01_hbm, t23_hbm]
    bufs = [rows_0, rows_1]
    sems = [sem_0, sem_1]
    sem_i = [sem_i0, sem_i1]
    sem_s = [sem_s0, sem_s1]
    srcs = [src0_v, src1_v]
    dsts = [dst0_v, dst1_v]
    qs = [q0_v, q1_v]
    ms = [m0_v, m1_v]
    sd = [sd0_v, sd1_v]

    def _idx_fetch(g, p):
        b = ebase + g * _B
        pltpu.async_copy(src_hbm.at[pl.ds(b, _B)], srcs[p], sem_i[p])
        pltpu.async_copy(dst_hbm.at[pl.ds(b, _B)], dsts[p], sem_i[p])
        pltpu.async_copy(q_hbm.at[pl.ds(b * _H, _B * _H)], qs[p], sem_i[p])

    def _idx_wait(p):
        pltpu.make_async_copy(src_hbm.at[pl.ds(0, _B)], srcs[p],
                              sem_i[p]).wait()
        pltpu.make_async_copy(dst_hbm.at[pl.ds(0, _B)], dsts[p],
                              sem_i[p]).wait()
        pltpu.make_async_copy(q_hbm.at[pl.ds(0, _B * _H)], qs[p],
                              sem_i[p]).wait()

    def _scatter_wait(p):
        pltpu.make_async_copy(ms[p], acc_sh.at[sd[p]], sem_s[p]).wait()

    def _chunk(g, p, prefetch, first):
        q_v = qs[p]
        m_v = ms[p]
        if prefetch:
            _idx_fetch(g + 1, 1 - p)
        for ph in range(2):
            pltpu.make_async_copy(tabs[ph].at[srcs[p]], bufs[ph],
                                  sems[ph]).wait()
            if ph == 0 and not first:
                _scatter_wait(p)
            buf = bufs[ph]

            def edge(e, ph=ph, buf=buf, q_v=q_v, m_v=m_v):
                qa = plsc.load_gather(q_v, [ones16 * (e * _H + 2 * ph)])
                qb = plsc.load_gather(q_v, [ones16 * (e * _H + 2 * ph + 1)])
                for j in range(_HID // 16):
                    a, b = plsc.unpack(
                        plsc.bitcast(buf[e, pl.ds(j * 16, 16)],
                                     jnp.bfloat16),
                        format=plsc.PackFormat.INTERLEAVED)
                    v = qa * a + qb * b
                    if ph > 0:
                        v = v + m_v[e, pl.ds(j * 16, 16)]
                    m_v[e, pl.ds(j * 16, 16)] = v

            plsc.parallel_loop(0, _B, unroll=2)(edge)
            if prefetch:
                if ph == 0:
                    _idx_wait(1 - p)
                pltpu.async_copy(tabs[ph].at[srcs[1 - p]], bufs[ph],
                                 sems[ph])
        for t in range(_B // 16):
            sd[p][pl.ds(t * 16, 16)] = dsts[p][pl.ds(t * 16, 16)]
        pltpu.async_copy(m_v, acc_sh.at[sd[p]], sem_s[p], add=True)

    _idx_fetch(0, 0)
    _idx_wait(0)
    pltpu.async_copy(tabs[0].at[srcs[0]], bufs[0], sems[0])
    pltpu.async_copy(tabs[1].at[srcs[0]], bufs[1], sems[1])

    _chunk(0, 0, True, True)
    _chunk(1, 1, True, True)

    def pair_body(i, carry):
        g = 2 + 2 * i
        _chunk(g, 0, True, False)
        _chunk(g + 1, 1, True, False)
        return carry

    lax.fori_loop(0, (_NCH - 3) // 2, pair_body, None)
    _chunk(_NCH - 1, (_NCH - 1) % 2, False, False)
    _scatter_wait(0)
    _scatter_wait(1)
    plsc.subcore_barrier()

    for k in range(_RPT // _B):
        r = row0 + k * _B
        pltpu.sync_copy(acc_sh.at[pl.ds(r, _B)],
                        out_hbm.at[pl.ds(cid * _N2 + r, _B)])


def _pack2(lo, hi):
    l16 = jax.lax.bitcast_convert_type(lo.astype(jnp.bfloat16),
                                       jnp.uint16).astype(jnp.uint32)
    h16 = jax.lax.bitcast_convert_type(hi.astype(jnp.bfloat16),
                                       jnp.uint16).astype(jnp.uint32)
    return jax.lax.bitcast_convert_type(l16 | (h16 << 16), jnp.float32)


def _head_tables(v, wc_ref, t01_ref, t23_ref):
    hd = [jnp.dot(v, wc_ref[:, i * _HID:(i + 1) * _HID],
                  preferred_element_type=jnp.float32) for i in range(_H)]
    t01_ref[...] = _pack2(hd[0], hd[1])
    t23_ref[...] = _pack2(hd[2], hd[3])


def _tc_pre_body(x_ref, w1_ref, b1_ref, w2_ref, b2_ref, u1_ref, wc_ref,
                 t01_ref, t23_ref, xu_ref, sk_ref):
    x = x_ref[...]
    xp = jnp.dot(x, w1_ref[...], preferred_element_type=jnp.float32) + b1_ref[...]
    _head_tables(xp, wc_ref, t01_ref, t23_ref)
    xu_ref[...] = jnp.dot(xp, u1_ref[...], preferred_element_type=jnp.float32)
    sk_ref[...] = jnp.dot(x, w2_ref[...], preferred_element_type=jnp.float32) + b2_ref[...]


def _tc_mid_body(a0_ref, a1_ref, cnt_ref, sk1_ref, bias1_ref, u2_ref,
                 wc_ref, w3_ref, b3_ref,
                 t01_ref, t23_ref, xu_ref, sk2_ref, inv_ref):
    s = a0_ref[...] + a1_ref[...]
    inv = 1.0 / jnp.maximum(cnt_ref[...][:, 0:1], 1.0)
    h = jnp.maximum(s * inv + bias1_ref[...] + sk1_ref[...], 0.0)
    _head_tables(h, wc_ref, t01_ref, t23_ref)
    xu_ref[...] = jnp.dot(h, u2_ref[...], preferred_element_type=jnp.float32)
    sk2_ref[...] = jnp.dot(h, w3_ref[...], preferred_element_type=jnp.float32) + b3_ref[...]
    inv_ref[...] = jnp.broadcast_to(inv, inv_ref.shape)


def _tc_post_body(a0_ref, a1_ref, sk2_ref, inv_ref, bias2_ref, out_ref):
    s = (a0_ref[...] + a1_ref[...]) * inv_ref[...][:, 0:1]
    out_ref[...] = jnp.maximum(s + bias2_ref[...] + sk2_ref[...], 0.0)


def _row_spec(w):
    return pl.BlockSpec((_RB, w), lambda i: (i, 0))


def _full_spec(r, c):
    return pl.BlockSpec((r, c), lambda i: (0, 0))


_GRID = (_N2 // _RB,)
_TABF = jax.ShapeDtypeStruct((_N2, _HID), jnp.float32)

_tc_pre = pl.pallas_call(
    _tc_pre_body,
    grid=_GRID,
    in_specs=[_row_spec(_D), _full_spec(_D, _D), _full_spec(1, _D),
              _full_spec(_D, _D), _full_spec(1, _D), _full_spec(_D, 16),
              _full_spec(_D, _XWW)],
    out_specs=[_row_spec(_HID)] * 2 + [_row_spec(16), _row_spec(_HID)],
    out_shape=[_TABF] * 2 + [jax.ShapeDtypeStruct((_N2, 16), jnp.float32),
                             _TABF],
)

_tc_mid = pl.pallas_call(
    _tc_mid_body,
    grid=_GRID,
    in_specs=[_row_spec(_HID), _row_spec(_HID), _row_spec(8), _row_spec(_HID),
              _full_spec(1, _HID), _full_spec(_HID, 16),
              _full_spec(_HID, _XWW),
              _full_spec(_HID, _HID), _full_spec(1, _HID)],
    out_specs=[_row_spec(_HID)] * 2 + [_row_spec(16), _row_spec(_HID),
                                       _row_spec(8)],
    out_shape=[_TABF] * 2 + [jax.ShapeDtypeStruct((_N2, 16), jnp.float32),
                             _TABF,
                             jax.ShapeDtypeStruct((_N2, 8), jnp.float32)],
)

_tc_post = pl.pallas_call(
    _tc_post_body,
    grid=_GRID,
    in_specs=[_row_spec(_HID), _row_spec(_HID), _row_spec(_HID),
              _row_spec(8), _full_spec(1, _HID)],
    out_specs=_row_spec(_HID),
    out_shape=jax.ShapeDtypeStruct((_N2, _HID), jnp.float32),
)


def kernel(x, edge_index, W1, b1, W2, b2, W3, b3, U1, c1, Wc1, bias1,
           U2, c2, Wc2, bias2):
    xpad = jnp.pad(x, ((0, _N2 - _N), (0, 0)))
    src = edge_index[0]
    dst = edge_index[1]
    u1p = jnp.pad(U1, ((0, 0), (0, 16 - _H)))
    u2p = jnp.pad(U2, ((0, 0), (0, 16 - _H)))
    c1p = jnp.pad(c1, (0, 16 - _H))
    c2p = jnp.pad(c2, (0, 16 - _H))

    t01a, t23a, xu1, sk1 = _tc_pre(
        xpad, W1, b1.reshape(1, -1), W2, b2.reshape(1, -1), u1p, Wc1)
    q1, cnt1 = _q_kernel_cnt(src, dst, xu1[:, :_H].reshape(-1), c1p)
    acc1, = _agg_kernel(src, dst, q1, t01a, t23a)
    cnt8 = jnp.broadcast_to(
        (cnt1[:_CR] + cnt1[_CR:]).reshape(_N2, 1), (_N2, 8))
    t01b, t23b, xu2, sk2, inv = _tc_mid(
        acc1[:_N2], acc1[_N2:], cnt8, sk1, bias1.reshape(1, -1), u2p,
        Wc2, W3, b3.reshape(1, -1))
    q2, = _q_kernel(src, dst, xu2[:, :_H].reshape(-1), c2p)
    acc2, = _agg_kernel(src, dst, q2, t01b, t23b)
    out = _tc_post(acc2[:_N2], acc2[_N2:], sk2, inv, bias2.reshape(1, -1))
    return out[:_N]
